# Initial kernel scaffold; baseline (speedup 1.0000x reference)
import jax
import jax.numpy as jnp
from jax.experimental import pallas as pl


def kernel(x, theta_w, theta_b, phi_w, phi_b, att_w1, att_b1, att_w2, att_b2, proj_w, proj_b, gamma, beta):
    raise NotImplementedError("write your pallas kernel here")


# speedup vs baseline: 14.1558x; 14.1558x over previous
import functools

import jax
import jax.numpy as jnp
from jax import lax
from jax.experimental import pallas as pl
from jax.experimental.pallas import tpu as pltpu
from jax.experimental.pallas import tpu_sc as plsc

B = 2
C_IN = 96
H = 64
W = 64
N = H * W
C_OUT = 96
RD = 24
K_SEM = 8
K_SP = 8
K_TOT = K_SP + K_SEM
HID = C_OUT // 2
NEG = -1e30

RT = 512
NT = N // RT
T4 = 256
TBL_W = C_OUT + HID

_OFFS = ((-1, -1), (-1, 0), (-1, 1), (0, -1), (0, 1), (1, -1), (1, 0), (1, 1))


def _k1_body(xf_ref, tw_ref, tb_ref, pw_ref, pb_ref, w1t_ref, w1b_ref, b1_ref,
             xn_ref, xnt_ref, tbl_ref, catt_ref):
    xf = xf_ref[...]
    tw = tw_ref[...]
    xr = lax.dot_general(tw, xf, (((1,), (0,)), ((), ())),
                         preferred_element_type=jnp.float32)
    xr = xr + tb_ref[...]
    xrt = lax.dot_general(xf, tw, (((0,), (1,)), ((), ())),
                          preferred_element_type=jnp.float32)
    xrt = xrt + tb_ref[...].reshape(1, RD)
    inv = lax.rsqrt(jnp.maximum(jnp.sum(xr * xr, axis=0, keepdims=True),
                                1e-24))
    invt = lax.rsqrt(jnp.maximum(jnp.sum(xrt * xrt, axis=1, keepdims=True),
                                 1e-24))
    xn_ref[...] = xr * inv
    xnt_ref[...] = xrt * invt
    xphi = lax.dot_general(xf, pw_ref[...], (((0,), (1,)), ((), ())),
                           preferred_element_type=jnp.float32)
    xphi = xphi + pb_ref[...]
    natt = jnp.dot(xphi, w1b_ref[...],
                   preferred_element_type=jnp.float32)
    catt = jnp.dot(xphi, w1t_ref[...],
                   preferred_element_type=jnp.float32) + b1_ref[...]
    tbl_ref[...] = jnp.concatenate([xphi, natt], axis=1)
    catt_ref[...] = catt


def _k1(xf, theta_w, theta_b, phi_w, phi_b, w1t, w1b, b1):
    return pl.pallas_call(
        _k1_body,
        grid=(B,),
        in_specs=[
            pl.BlockSpec((None, C_IN, N), lambda b: (b, 0, 0)),
            pl.BlockSpec((RD, C_IN), lambda b: (0, 0)),
            pl.BlockSpec((RD, 1), lambda b: (0, 0)),
            pl.BlockSpec((C_OUT, C_IN), lambda b: (0, 0)),
            pl.BlockSpec((1, C_OUT), lambda b: (0, 0)),
            pl.BlockSpec((C_OUT, HID), lambda b: (0, 0)),
            pl.BlockSpec((C_OUT, HID), lambda b: (0, 0)),
            pl.BlockSpec((1, HID), lambda b: (0, 0)),
        ],
        out_specs=[
            pl.BlockSpec((None, RD, N), lambda b: (b, 0, 0)),
            pl.BlockSpec((None, N, RD), lambda b: (b, 0, 0)),
            pl.BlockSpec((None, N, TBL_W), lambda b: (b, 0, 0)),
            pl.BlockSpec((None, N, HID), lambda b: (b, 0, 0)),
        ],
        out_shape=[
            jax.ShapeDtypeStruct((B, RD, N), jnp.float32),
            jax.ShapeDtypeStruct((B, N, RD), jnp.float32),
            jax.ShapeDtypeStruct((B, N, TBL_W), jnp.float32),
            jax.ShapeDtypeStruct((B, N, HID), jnp.float32),
        ],
    )(xf, theta_w, theta_b, phi_w, phi_b, w1t, w1b, b1)


def _k2_body(lhs_ref, rhs_ref, out_ref):
    b = pl.program_id(0)
    t = pl.program_id(1)
    sim = lax.dot_general(lhs_ref[...], rhs_ref[...],
                          (((1,), (0,)), ((), ())),
                          preferred_element_type=jnp.float32)
    r = t * RT + lax.broadcasted_iota(jnp.int32, (RT, N), 0)
    c = lax.broadcasted_iota(jnp.int32, (RT, N), 1)
    ri, rj = r // W, r % W
    ci, cj = c // W, c % W
    spatial = (jnp.abs(ri - ci) <= 1) & (jnp.abs(rj - cj) <= 1)
    sim = jnp.where(spatial, NEG, sim)
    cols = []
    r1 = t * RT + lax.broadcasted_iota(jnp.int32, (RT, 1), 0)
    ri1, rj1 = r1 // W, r1 % W
    for di, dj in _OFFS:
        ni = jnp.clip(ri1 + di, 0, H - 1)
        nj = jnp.clip(rj1 + dj, 0, W - 1)
        cols.append(ni * W + nj)
    for _ in range(K_SEM):
        m = jnp.max(sim, axis=1, keepdims=True)
        idx = jnp.min(jnp.where(sim == m, c, N), axis=1, keepdims=True)
        cols.append(idx)
        sim = jnp.where(c == idx, NEG, sim)
    out_ref[...] = jnp.concatenate(cols, axis=1) + b * N


def _k2(xnt, xn):
    return pl.pallas_call(
        _k2_body,
        grid=(B, NT),
        in_specs=[
            pl.BlockSpec((None, RT, RD), lambda b, t: (b, t, 0)),
            pl.BlockSpec((None, RD, N), lambda b, t: (b, 0, 0)),
        ],
        out_specs=pl.BlockSpec((None, RT, K_TOT), lambda b, t: (b, t, 0)),
        out_shape=jax.ShapeDtypeStruct((B, N, K_TOT), jnp.int32),
    )(xnt, xn)


_N_EDGE = B * N * K_TOT
_NW = 32
_CH = 128
_NCH = _N_EDGE // (_NW * _CH)


def _sc_gather(tbl2, idx2):
    mesh = plsc.VectorSubcoreMesh(core_axis_name="c", subcore_axis_name="s")

    @functools.partial(
        pl.kernel,
        out_type=jax.ShapeDtypeStruct((_N_EDGE, TBL_W), jnp.float32),
        mesh=mesh,
        scratch_types=[
            pltpu.VMEM((_NCH, _CH), jnp.int32),
            pltpu.VMEM((_CH, TBL_W), jnp.float32),
            pltpu.SemaphoreType.DMA,
        ],
        compiler_params=pltpu.CompilerParams(use_tc_tiling_on_sc=False),
    )
    def k(tbl_hbm, idx_hbm, out_hbm, idx_v, buf, sem):
        wid = lax.axis_index("s") * 2 + lax.axis_index("c")
        pltpu.sync_copy(idx_hbm.at[pl.ds(wid * _NCH, _NCH)], idx_v)

        def body(ci, carry):
            pltpu.async_copy(tbl_hbm.at[idx_v.at[ci]], buf, sem).wait()
            pltpu.sync_copy(
                buf, out_hbm.at[pl.ds(wid * _NCH * _CH + ci * _CH, _CH)])
            return carry

        lax.fori_loop(0, _NCH, body, 0)

    return k(tbl2, idx2)


def _k4_body(g_ref, ca_ref, w2_ref, agg_ref, sa_ref, gm_ref):
    t = pl.program_id(0)
    g = g_ref[...]
    xnb = g[:, :, :C_OUT]
    na = g[:, :, C_OUT:]
    h = jnp.maximum(na + ca_ref[...][:, None, :], 0.0)
    logit = jnp.sum(h * w2_ref[...], axis=2)
    m = jnp.max(logit, axis=1, keepdims=True)
    e = jnp.exp(logit - m)
    w = e / jnp.sum(e, axis=1, keepdims=True)
    agg = jnp.sum(xnb * w[:, :, None], axis=1)
    agg_ref[...] = agg

    @pl.when(t == 0)
    def _():
        sa_ref[...] = jnp.zeros_like(sa_ref)
        gm_ref[...] = jnp.zeros_like(gm_ref)

    sa_ref[...] += jnp.broadcast_to(jnp.sum(agg, axis=0, keepdims=True),
                                    (8, C_OUT))
    gm_ref[...] += lax.dot_general(agg, agg, (((0,), (0,)), ((), ())),
                                   preferred_element_type=jnp.float32)


def _k4(gath3, catt2, w2r):
    n_nodes = B * N
    return pl.pallas_call(
        _k4_body,
        grid=(n_nodes // T4,),
        in_specs=[
            pl.BlockSpec((T4, K_TOT, TBL_W), lambda t: (t, 0, 0)),
            pl.BlockSpec((T4, HID), lambda t: (t, 0)),
            pl.BlockSpec((1, 1, HID), lambda t: (0, 0, 0)),
        ],
        out_specs=[
            pl.BlockSpec((T4, C_OUT), lambda t: (t, 0)),
            pl.BlockSpec((8, C_OUT), lambda t: (0, 0)),
            pl.BlockSpec((C_OUT, C_OUT), lambda t: (0, 0)),
        ],
        out_shape=[
            jax.ShapeDtypeStruct((n_nodes, C_OUT), jnp.float32),
            jax.ShapeDtypeStruct((8, C_OUT), jnp.float32),
            jax.ShapeDtypeStruct((C_OUT, C_OUT), jnp.float32),
        ],
    )(gath3, catt2, w2r)


def _k5_body(agg_ref, sa_ref, gm_ref, pw_ref, pb_ref, gam_ref, bet_ref,
             out_ref):
    inv_n = 1.0 / (B * N)
    mu = sa_ref[...][0:1, :] * inv_n
    egg = gm_ref[...] * inv_n
    cov = egg - lax.dot_general(mu, mu, (((0,), (0,)), ((), ())),
                                preferred_element_type=jnp.float32)
    pw = pw_ref[...]
    mcov = jnp.dot(pw, cov, preferred_element_type=jnp.float32)
    dmat = lax.dot_general(mcov, pw, (((1,), (1,)), ((), ())),
                           preferred_element_type=jnp.float32)
    eye = (lax.broadcasted_iota(jnp.int32, (C_OUT, C_OUT), 0) ==
           lax.broadcasted_iota(jnp.int32, (C_OUT, C_OUT), 1))
    var = jnp.sum(jnp.where(eye, dmat, 0.0), axis=0, keepdims=True)
    mean_o = lax.dot_general(mu, pw, (((1,), (1,)), ((), ())),
                             preferred_element_type=jnp.float32) \
        + pb_ref[...]
    scale = lax.rsqrt(var + 1e-5) * gam_ref[...]
    o = lax.dot_general(agg_ref[...], pw, (((1,), (1,)), ((), ())),
                        preferred_element_type=jnp.float32)
    o = (o + pb_ref[...] - mean_o) * scale + bet_ref[...]
    out_ref[...] = jnp.maximum(o, 0.0)


def _k5(agg3, sa, gm, proj_w, proj_b, gamma, beta):
    return pl.pallas_call(
        _k5_body,
        grid=(B, NT),
        in_specs=[
            pl.BlockSpec((None, RT, C_OUT), lambda b, t: (b, t, 0)),
            pl.BlockSpec((8, C_OUT), lambda b, t: (0, 0)),
            pl.BlockSpec((C_OUT, C_OUT), lambda b, t: (0, 0)),
            pl.BlockSpec((C_OUT, C_OUT), lambda b, t: (0, 0)),
            pl.BlockSpec((1, C_OUT), lambda b, t: (0, 0)),
            pl.BlockSpec((1, C_OUT), lambda b, t: (0, 0)),
            pl.BlockSpec((1, C_OUT), lambda b, t: (0, 0)),
        ],
        out_specs=pl.BlockSpec((None, RT, C_OUT), lambda b, t: (b, t, 0)),
        out_shape=jax.ShapeDtypeStruct((B, N, C_OUT), jnp.float32),
    )(agg3, sa, gm, proj_w, proj_b, gamma, beta)


def kernel(x, theta_w, theta_b, phi_w, phi_b, att_w1, att_b1, att_w2, att_b2,
           proj_w, proj_b, gamma, beta):
    xf = x.reshape(B, C_IN, N)
    w1t = att_w1[:C_OUT]
    w1b = att_w1[C_OUT:]
    xn, xnt, tbl, catt = _k1(
        xf, theta_w, theta_b.reshape(RD, 1), phi_w, phi_b.reshape(1, C_OUT),
        w1t, w1b, att_b1.reshape(1, HID))
    gidx = _k2(xnt, xn)
    gath = _sc_gather(tbl.reshape(B * N, TBL_W),
                      gidx.reshape(_N_EDGE // _CH, _CH))
    agg, sa, gm = _k4(gath.reshape(B * N, K_TOT, TBL_W),
                      catt.reshape(B * N, HID),
                      att_w2.reshape(1, 1, HID))
    out = _k5(agg.reshape(B, N, C_OUT), sa, gm, proj_w,
              proj_b.reshape(1, C_OUT), gamma.reshape(1, C_OUT),
              beta.reshape(1, C_OUT))
    return out.transpose(0, 2, 1).reshape(B, C_OUT, H, W)


# speedup vs baseline: 14.9662x; 1.0572x over previous
import functools

import jax
import jax.numpy as jnp
from jax import lax
from jax.experimental import pallas as pl
from jax.experimental.pallas import tpu as pltpu
from jax.experimental.pallas import tpu_sc as plsc

B = 2
C_IN = 96
H = 64
W = 64
N = H * W
C_OUT = 96
RD = 24
K_SEM = 8
K_SP = 8
K_TOT = K_SP + K_SEM
HID = C_OUT // 2
NEG = -1e30

RT = 256
NT = N // RT
T4 = 512
TBL_W = C_OUT + HID

_OFFS = ((-1, -1), (-1, 0), (-1, 1), (0, -1), (0, 1), (1, -1), (1, 0), (1, 1))


def _k1_body(xf_ref, tw_ref, tb_ref, pw_ref, pb_ref, w1t_ref, w1b_ref, b1_ref,
             xn_ref, xnt_ref, tbl_ref, catt_ref):
    xf = xf_ref[...]
    tw = tw_ref[...]
    xr = lax.dot_general(tw, xf, (((1,), (0,)), ((), ())),
                         preferred_element_type=jnp.float32)
    xr = xr + tb_ref[...]
    xrt = lax.dot_general(xf, tw, (((0,), (1,)), ((), ())),
                          preferred_element_type=jnp.float32)
    xrt = xrt + tb_ref[...].reshape(1, RD)
    inv = lax.rsqrt(jnp.maximum(jnp.sum(xr * xr, axis=0, keepdims=True),
                                1e-24))
    invt = lax.rsqrt(jnp.maximum(jnp.sum(xrt * xrt, axis=1, keepdims=True),
                                 1e-24))
    xn_ref[...] = xr * inv
    xnt_ref[...] = xrt * invt
    xphi = lax.dot_general(xf, pw_ref[...], (((0,), (1,)), ((), ())),
                           preferred_element_type=jnp.float32)
    xphi = xphi + pb_ref[...]
    natt = jnp.dot(xphi, w1b_ref[...],
                   preferred_element_type=jnp.float32)
    catt = jnp.dot(xphi, w1t_ref[...],
                   preferred_element_type=jnp.float32) + b1_ref[...]
    tbl_ref[...] = jnp.concatenate([xphi, natt], axis=1)
    catt_ref[...] = catt


def _k1(xf, theta_w, theta_b, phi_w, phi_b, w1t, w1b, b1):
    return pl.pallas_call(
        _k1_body,
        grid=(B,),
        in_specs=[
            pl.BlockSpec((None, C_IN, N), lambda b: (b, 0, 0)),
            pl.BlockSpec((RD, C_IN), lambda b: (0, 0)),
            pl.BlockSpec((RD, 1), lambda b: (0, 0)),
            pl.BlockSpec((C_OUT, C_IN), lambda b: (0, 0)),
            pl.BlockSpec((1, C_OUT), lambda b: (0, 0)),
            pl.BlockSpec((C_OUT, HID), lambda b: (0, 0)),
            pl.BlockSpec((C_OUT, HID), lambda b: (0, 0)),
            pl.BlockSpec((1, HID), lambda b: (0, 0)),
        ],
        out_specs=[
            pl.BlockSpec((None, RD, N), lambda b: (b, 0, 0)),
            pl.BlockSpec((None, N, RD), lambda b: (b, 0, 0)),
            pl.BlockSpec((None, N, TBL_W), lambda b: (b, 0, 0)),
            pl.BlockSpec((None, N, HID), lambda b: (b, 0, 0)),
        ],
        out_shape=[
            jax.ShapeDtypeStruct((B, RD, N), jnp.float32),
            jax.ShapeDtypeStruct((B, N, RD), jnp.float32),
            jax.ShapeDtypeStruct((B, N, TBL_W), jnp.float32),
            jax.ShapeDtypeStruct((B, N, HID), jnp.float32),
        ],
    )(xf, theta_w, theta_b, phi_w, phi_b, w1t, w1b, b1)


def _k2_body(lhs_ref, rhs_ref, out_ref):
    b = pl.program_id(0)
    t = pl.program_id(1)
    sim = lax.dot_general(lhs_ref[...], rhs_ref[...],
                          (((1,), (0,)), ((), ())),
                          preferred_element_type=jnp.float32)
    r1 = t * RT + lax.broadcasted_iota(jnp.int32, (RT, 1), 0)
    ri1, rj1 = r1 // W, r1 % W
    c1 = lax.broadcasted_iota(jnp.int32, (1, N), 1)
    ci1, cj1 = c1 // W, c1 % W
    spatial = ((jnp.abs(ri1 - ci1) <= 1) &
               (jnp.abs(rj1 - cj1) <= 1))
    sim = jnp.where(spatial, NEG, sim)
    cols = []
    for di, dj in _OFFS:
        ni = jnp.clip(ri1 + di, 0, H - 1)
        nj = jnp.clip(rj1 + dj, 0, W - 1)
        cols.append(ni * W + nj)
    cf = lax.broadcasted_iota(jnp.int32, (1, N), 1).astype(jnp.float32)
    for _ in range(K_SEM):
        m = jnp.max(sim, axis=1, keepdims=True)
        eq = sim == m
        idx = jnp.min(jnp.where(eq, cf, 1e9), axis=1, keepdims=True)
        cols.append(idx.astype(jnp.int32))
        sim = jnp.where(eq, NEG, sim)
    out_ref[...] = jnp.concatenate(cols, axis=1) + b * N


def _k2(xnt, xn):
    return pl.pallas_call(
        _k2_body,
        grid=(B, NT),
        in_specs=[
            pl.BlockSpec((None, RT, RD), lambda b, t: (b, t, 0)),
            pl.BlockSpec((None, RD, N), lambda b, t: (b, 0, 0)),
        ],
        out_specs=pl.BlockSpec((None, RT, K_TOT), lambda b, t: (b, t, 0)),
        out_shape=jax.ShapeDtypeStruct((B, N, K_TOT), jnp.int32),
    )(xnt, xn)


_N_EDGE = B * N * K_TOT
_NW = 32
_CH = 128
_NCH = _N_EDGE // (_NW * _CH)


def _sc_gather(tbl2, idx2):
    mesh = plsc.VectorSubcoreMesh(core_axis_name="c", subcore_axis_name="s")

    @functools.partial(
        pl.kernel,
        out_type=jax.ShapeDtypeStruct((_N_EDGE, TBL_W), jnp.float32),
        mesh=mesh,
        scratch_types=[
            pltpu.VMEM((_NCH, _CH), jnp.int32),
            pltpu.VMEM((_CH, TBL_W), jnp.float32),
            pltpu.SemaphoreType.DMA,
        ],
        compiler_params=pltpu.CompilerParams(use_tc_tiling_on_sc=False),
    )
    def k(tbl_hbm, idx_hbm, out_hbm, idx_v, buf, sem):
        wid = lax.axis_index("s") * 2 + lax.axis_index("c")
        pltpu.sync_copy(idx_hbm.at[pl.ds(wid * _NCH, _NCH)], idx_v)

        def body(ci, carry):
            pltpu.async_copy(tbl_hbm.at[idx_v.at[ci]], buf, sem).wait()
            pltpu.sync_copy(
                buf, out_hbm.at[pl.ds(wid * _NCH * _CH + ci * _CH, _CH)])
            return carry

        lax.fori_loop(0, _NCH, body, 0)

    return k(tbl2, idx2)


def _k4_body(g_ref, ca_ref, w2_ref, agg_ref, sa_ref, gm_ref):
    t = pl.program_id(0)
    g = g_ref[...]
    xnb = g[:, :, :C_OUT]
    na = g[:, :, C_OUT:]
    h = jnp.maximum(na + ca_ref[...][:, None, :], 0.0)
    logit = jnp.sum(h * w2_ref[...], axis=2)
    m = jnp.max(logit, axis=1, keepdims=True)
    e = jnp.exp(logit - m)
    w = e / jnp.sum(e, axis=1, keepdims=True)
    agg = jnp.sum(xnb * w[:, :, None], axis=1)
    agg_ref[...] = agg

    @pl.when(t == 0)
    def _():
        sa_ref[...] = jnp.zeros_like(sa_ref)
        gm_ref[...] = jnp.zeros_like(gm_ref)

    sa_ref[...] += jnp.broadcast_to(jnp.sum(agg, axis=0, keepdims=True),
                                    (8, C_OUT))
    gm_ref[...] += lax.dot_general(agg, agg, (((0,), (0,)), ((), ())),
                                   preferred_element_type=jnp.float32)


def _k4(gath3, catt2, w2r):
    n_nodes = B * N
    return pl.pallas_call(
        _k4_body,
        grid=(n_nodes // T4,),
        in_specs=[
            pl.BlockSpec((T4, K_TOT, TBL_W), lambda t: (t, 0, 0)),
            pl.BlockSpec((T4, HID), lambda t: (t, 0)),
            pl.BlockSpec((1, 1, HID), lambda t: (0, 0, 0)),
        ],
        out_specs=[
            pl.BlockSpec((T4, C_OUT), lambda t: (t, 0)),
            pl.BlockSpec((8, C_OUT), lambda t: (0, 0)),
            pl.BlockSpec((C_OUT, C_OUT), lambda t: (0, 0)),
        ],
        out_shape=[
            jax.ShapeDtypeStruct((n_nodes, C_OUT), jnp.float32),
            jax.ShapeDtypeStruct((8, C_OUT), jnp.float32),
            jax.ShapeDtypeStruct((C_OUT, C_OUT), jnp.float32),
        ],
    )(gath3, catt2, w2r)


def _k5_body(agg_ref, sa_ref, gm_ref, pw_ref, pb_ref, gam_ref, bet_ref,
             out_ref):
    inv_n = 1.0 / (B * N)
    mu = sa_ref[...][0:1, :] * inv_n
    egg = gm_ref[...] * inv_n
    cov = egg - lax.dot_general(mu, mu, (((0,), (0,)), ((), ())),
                                preferred_element_type=jnp.float32)
    pw = pw_ref[...]
    mcov = jnp.dot(pw, cov, preferred_element_type=jnp.float32)
    dmat = lax.dot_general(mcov, pw, (((1,), (1,)), ((), ())),
                           preferred_element_type=jnp.float32)
    eye = (lax.broadcasted_iota(jnp.int32, (C_OUT, C_OUT), 0) ==
           lax.broadcasted_iota(jnp.int32, (C_OUT, C_OUT), 1))
    var = jnp.sum(jnp.where(eye, dmat, 0.0), axis=0, keepdims=True)
    mean_o = lax.dot_general(mu, pw, (((1,), (1,)), ((), ())),
                             preferred_element_type=jnp.float32) \
        + pb_ref[...]
    scale = lax.rsqrt(var + 1e-5) * gam_ref[...]
    o = lax.dot_general(agg_ref[...], pw, (((1,), (1,)), ((), ())),
                        preferred_element_type=jnp.float32)
    o = (o + pb_ref[...] - mean_o) * scale + bet_ref[...]
    out_ref[...] = jnp.maximum(o, 0.0)


def _k5(agg3, sa, gm, proj_w, proj_b, gamma, beta):
    return pl.pallas_call(
        _k5_body,
        grid=(B, NT),
        in_specs=[
            pl.BlockSpec((None, RT, C_OUT), lambda b, t: (b, t, 0)),
            pl.BlockSpec((8, C_OUT), lambda b, t: (0, 0)),
            pl.BlockSpec((C_OUT, C_OUT), lambda b, t: (0, 0)),
            pl.BlockSpec((C_OUT, C_OUT), lambda b, t: (0, 0)),
            pl.BlockSpec((1, C_OUT), lambda b, t: (0, 0)),
            pl.BlockSpec((1, C_OUT), lambda b, t: (0, 0)),
            pl.BlockSpec((1, C_OUT), lambda b, t: (0, 0)),
        ],
        out_specs=pl.BlockSpec((None, RT, C_OUT), lambda b, t: (b, t, 0)),
        out_shape=jax.ShapeDtypeStruct((B, N, C_OUT), jnp.float32),
    )(agg3, sa, gm, proj_w, proj_b, gamma, beta)


def kernel(x, theta_w, theta_b, phi_w, phi_b, att_w1, att_b1, att_w2, att_b2,
           proj_w, proj_b, gamma, beta):
    xf = x.reshape(B, C_IN, N)
    w1t = att_w1[:C_OUT]
    w1b = att_w1[C_OUT:]
    xn, xnt, tbl, catt = _k1(
        xf, theta_w, theta_b.reshape(RD, 1), phi_w, phi_b.reshape(1, C_OUT),
        w1t, w1b, att_b1.reshape(1, HID))
    gidx = _k2(xnt, xn)
    gath = _sc_gather(tbl.reshape(B * N, TBL_W),
                      gidx.reshape(_N_EDGE // _CH, _CH))
    agg, sa, gm = _k4(gath.reshape(B * N, K_TOT, TBL_W),
                      catt.reshape(B * N, HID),
                      att_w2.reshape(1, 1, HID))
    out = _k5(agg.reshape(B, N, C_OUT), sa, gm, proj_w,
              proj_b.reshape(1, C_OUT), gamma.reshape(1, C_OUT),
              beta.reshape(1, C_OUT))
    return out.transpose(0, 2, 1).reshape(B, C_OUT, H, W)
